# baseline (device time: 66147 ns/iter reference)
import jax
import jax.numpy as jnp
from jax import lax
from jax.experimental import pallas as pl
from jax.experimental.pallas import tpu as pltpu

N = 32
B = 2
SQ = 512
SKV = 512
HL = 8
DH = 64
DM = 768
HD = HL * DH
ROWS = B * SQ
CHUNK = ROWS // N
WIN = 128
WAVE = 8
DO_COMM = True


def kernel(x, Wq, K_ext, V_ext, Wo):
    K_t = jnp.transpose(K_ext, (0, 2, 1, 3))
    V_t = jnp.transpose(V_ext, (0, 2, 1, 3))

    def body(x_ref, wq_ref, k_ref, v_ref, wo_ref, out_ref,
             wq_s, wo_s, p_ref, g_ref, rs_buf,
             pre_sems, rs_send, rs_recv, ag_send, ag_recv):
        me = lax.axis_index("i")

        if DO_COMM:
            barrier_sem = pltpu.get_barrier_semaphore()
            for j in range(1, N):
                pl.semaphore_signal(
                    barrier_sem, inc=1,
                    device_id=(lax.rem(me + j, N),),
                    device_id_type=pl.DeviceIdType.MESH)

        col0 = me * HD
        cp_wq = pltpu.make_async_copy(
            wq_ref.at[:, pl.ds(col0, HD)], wq_s, pre_sems.at[0])
        cp_wo = pltpu.make_async_copy(
            wo_ref.at[pl.ds(col0, HD), :], wo_s, pre_sems.at[1])
        cp_wq.start()
        cp_wo.start()

        qi = lax.broadcasted_iota(jnp.int32, (SQ, SKV), 0)
        ki = lax.broadcasted_iota(jnp.int32, (SQ, SKV), 1)
        mask = jnp.abs(qi - ki) <= WIN

        cp_wq.wait()
        cp_wo.wait()

        bf = jnp.bfloat16
        f32 = jnp.float32
        wq16 = wq_s[...].astype(bf)
        wo16 = wo_s[...].astype(bf)
        for b in range(B):
            q2 = jnp.dot(x_ref[b].astype(bf), wq16,
                         preferred_element_type=f32)
            ctx_cols = []
            for h in range(HL):
                q = q2[:, h * DH:(h + 1) * DH].astype(bf)
                k = k_ref[b, h].astype(bf)
                v = v_ref[b, h].astype(bf)
                s = lax.dot_general(
                    q, k, (((1,), (1,)), ((), ())),
                    preferred_element_type=f32) * 0.125
                s = jnp.where(mask, s, -1e9)
                m = jnp.max(s, axis=1, keepdims=True)
                w = jnp.exp(s - m)
                w = w / jnp.sum(w, axis=1, keepdims=True)
                ctx_cols.append(jnp.dot(w.astype(bf), v,
                                        preferred_element_type=f32))
            ctx2 = jnp.concatenate(ctx_cols, axis=1)
            p_ref[pl.ds(b * SQ, SQ), :] = jnp.dot(
                ctx2.astype(bf), wo16,
                preferred_element_type=f32).astype(bf)

        if not DO_COMM:
            out_ref[...] = p_ref[...].astype(f32)
            return

        pl.semaphore_wait(barrier_sem, N - 1)

        waves = [list(range(w0, min(w0 + WAVE, N)))
                 for w0 in range(1, N, WAVE)]

        def start_rs_wave(wave):
            rdmas = []
            for j in wave:
                dest = lax.rem(me + j, N)
                slot = (N - 1) - j
                rdma = pltpu.make_async_remote_copy(
                    src_ref=p_ref.at[pl.ds(dest * CHUNK, CHUNK), :],
                    dst_ref=rs_buf.at[slot],
                    send_sem=rs_send.at[j - 1],
                    recv_sem=rs_recv.at[slot],
                    device_id=(dest,),
                    device_id_type=pl.DeviceIdType.MESH,
                )
                rdma.start()
                rdmas.append((j, rdma))
            return rdmas

        acc = p_ref[pl.ds(me * CHUNK, CHUNK), :].astype(f32)
        inflight = start_rs_wave(waves[0])
        for w in range(len(waves)):
            nxt = start_rs_wave(waves[w + 1]) if w + 1 < len(waves) else None
            for j, rdma in inflight:
                rdma.wait_recv()
                acc = acc + rs_buf[(N - 1) - j].astype(f32)
            for _, rdma in inflight:
                rdma.wait_send()
            inflight = nxt
        g_ref[pl.ds(me * CHUNK, CHUNK), :] = acc.astype(bf)
        out_ref[pl.ds(me * CHUNK, CHUNK), :] = acc

        def start_ag_wave(wave):
            rdmas = []
            for j in wave:
                dest = lax.rem(me + j, N)
                rdma = pltpu.make_async_remote_copy(
                    src_ref=g_ref.at[pl.ds(me * CHUNK, CHUNK), :],
                    dst_ref=g_ref.at[pl.ds(me * CHUNK, CHUNK), :],
                    send_sem=ag_send.at[j - 1],
                    recv_sem=ag_recv.at[(N - 1) - j],
                    device_id=(dest,),
                    device_id_type=pl.DeviceIdType.MESH,
                )
                rdma.start()
                rdmas.append((j, rdma))
            return rdmas

        inflight = start_ag_wave(waves[0])
        for w in range(len(waves)):
            nxt = start_ag_wave(waves[w + 1]) if w + 1 < len(waves) else None
            for j, rdma in inflight:
                rdma.wait_recv()
                c = lax.rem(me - j + N, N)
                out_ref[pl.ds(c * CHUNK, CHUNK), :] = (
                    g_ref[pl.ds(c * CHUNK, CHUNK), :].astype(f32))
            for _, rdma in inflight:
                rdma.wait_send()
            inflight = nxt

    out_flat = pl.pallas_call(
        body,
        out_shape=jax.ShapeDtypeStruct((ROWS, DM), jnp.float32),
        in_specs=[
            pl.BlockSpec(memory_space=pltpu.VMEM),
            pl.BlockSpec(memory_space=pl.ANY),
            pl.BlockSpec(memory_space=pltpu.VMEM),
            pl.BlockSpec(memory_space=pltpu.VMEM),
            pl.BlockSpec(memory_space=pl.ANY),
        ],
        out_specs=pl.BlockSpec(memory_space=pltpu.VMEM),
        scratch_shapes=[
            pltpu.VMEM((DM, HD), jnp.float32),
            pltpu.VMEM((HD, DM), jnp.float32),
            pltpu.VMEM((ROWS, DM), jnp.bfloat16),
            pltpu.VMEM((ROWS, DM), jnp.bfloat16),
            pltpu.VMEM((N - 1, CHUNK, DM), jnp.bfloat16),
            pltpu.SemaphoreType.DMA((2,)),
            pltpu.SemaphoreType.DMA((N - 1,)),
            pltpu.SemaphoreType.DMA((N - 1,)),
            pltpu.SemaphoreType.DMA((N - 1,)),
            pltpu.SemaphoreType.DMA((N - 1,)),
        ],
        compiler_params=(
            pltpu.CompilerParams(collective_id=0) if DO_COMM
            else pltpu.CompilerParams()),
    )(x, Wq, K_t, V_t, Wo)
    return out_flat.reshape(B, SQ, DM)


# device time: 64179 ns/iter; 1.0307x vs baseline; 1.0307x over previous
import jax
import jax.numpy as jnp
from jax import lax
from jax.experimental import pallas as pl
from jax.experimental.pallas import tpu as pltpu

N = 32
B = 2
SQ = 512
SKV = 512
HL = 8
DH = 64
DM = 768
HD = HL * DH
ROWS = B * SQ
CHUNK = ROWS // N
WIN = 128
WAVE = 11
DO_COMM = True


def kernel(x, Wq, K_ext, V_ext, Wo):
    K_t = jnp.transpose(K_ext, (0, 2, 1, 3))
    V_t = jnp.transpose(V_ext, (0, 2, 1, 3))

    def body(x_ref, wq_ref, k_ref, v_ref, wo_ref, out_ref,
             wq_s, wo_s, p_ref, g_ref, rs_buf,
             pre_sems, rs_send, rs_recv, ag_send, ag_recv):
        me = lax.axis_index("i")

        if DO_COMM:
            barrier_sem = pltpu.get_barrier_semaphore()
            for j in range(1, N):
                pl.semaphore_signal(
                    barrier_sem, inc=1,
                    device_id=(lax.rem(me + j, N),),
                    device_id_type=pl.DeviceIdType.MESH)

        col0 = me * HD
        cp_wq = pltpu.make_async_copy(
            wq_ref.at[:, pl.ds(col0, HD)], wq_s, pre_sems.at[0])
        cp_wo = pltpu.make_async_copy(
            wo_ref.at[pl.ds(col0, HD), :], wo_s, pre_sems.at[1])
        cp_wq.start()
        cp_wo.start()

        qi = lax.broadcasted_iota(jnp.int32, (SQ, SKV), 0)
        ki = lax.broadcasted_iota(jnp.int32, (SQ, SKV), 1)
        mask = jnp.abs(qi - ki) <= WIN

        cp_wq.wait()
        cp_wo.wait()

        bf = jnp.bfloat16
        f32 = jnp.float32
        wq16 = wq_s[...].astype(bf)
        wo16 = wo_s[...].astype(bf)
        for b in range(B):
            q2 = jnp.dot(x_ref[b].astype(bf), wq16,
                         preferred_element_type=f32)
            ctx_cols = []
            for h in range(HL):
                q = q2[:, h * DH:(h + 1) * DH].astype(bf)
                k = k_ref[b, h].astype(bf)
                v = v_ref[b, h].astype(bf)
                s = lax.dot_general(
                    q, k, (((1,), (1,)), ((), ())),
                    preferred_element_type=f32) * 0.125
                s = jnp.where(mask, s, -1e9)
                m = jnp.max(s, axis=1, keepdims=True)
                w = jnp.exp(s - m)
                w = w / jnp.sum(w, axis=1, keepdims=True)
                ctx_cols.append(jnp.dot(w.astype(bf), v,
                                        preferred_element_type=f32))
            ctx2 = jnp.concatenate(ctx_cols, axis=1)
            p_ref[pl.ds(b * SQ, SQ), :] = jnp.dot(
                ctx2.astype(bf), wo16,
                preferred_element_type=f32).astype(bf)

        if not DO_COMM:
            out_ref[...] = p_ref[...].astype(f32)
            return

        pl.semaphore_wait(barrier_sem, N - 1)

        waves = [list(range(w0, min(w0 + WAVE, N)))
                 for w0 in range(1, N, WAVE)]

        def start_rs_wave(wave):
            rdmas = []
            for j in wave:
                dest = lax.rem(me + j, N)
                slot = (N - 1) - j
                rdma = pltpu.make_async_remote_copy(
                    src_ref=p_ref.at[pl.ds(dest * CHUNK, CHUNK), :],
                    dst_ref=rs_buf.at[slot],
                    send_sem=rs_send.at[j - 1],
                    recv_sem=rs_recv.at[slot],
                    device_id=(dest,),
                    device_id_type=pl.DeviceIdType.MESH,
                )
                rdma.start()
                rdmas.append((j, rdma))
            return rdmas

        acc = p_ref[pl.ds(me * CHUNK, CHUNK), :].astype(f32)
        inflight = start_rs_wave(waves[0])
        for w in range(len(waves)):
            nxt = start_rs_wave(waves[w + 1]) if w + 1 < len(waves) else None
            for j, rdma in inflight:
                rdma.wait_recv()
                acc = acc + rs_buf[(N - 1) - j].astype(f32)
            for _, rdma in inflight:
                rdma.wait_send()
            inflight = nxt
        g_ref[pl.ds(me * CHUNK, CHUNK), :] = acc.astype(bf)
        out_ref[pl.ds(me * CHUNK, CHUNK), :] = acc

        def start_ag_wave(wave):
            rdmas = []
            for j in wave:
                dest = lax.rem(me + j, N)
                rdma = pltpu.make_async_remote_copy(
                    src_ref=g_ref.at[pl.ds(me * CHUNK, CHUNK), :],
                    dst_ref=g_ref.at[pl.ds(me * CHUNK, CHUNK), :],
                    send_sem=ag_send.at[j - 1],
                    recv_sem=ag_recv.at[(N - 1) - j],
                    device_id=(dest,),
                    device_id_type=pl.DeviceIdType.MESH,
                )
                rdma.start()
                rdmas.append((j, rdma))
            return rdmas

        inflight = start_ag_wave(waves[0])
        for w in range(len(waves)):
            nxt = start_ag_wave(waves[w + 1]) if w + 1 < len(waves) else None
            for j, rdma in inflight:
                rdma.wait_recv()
                c = lax.rem(me - j + N, N)
                out_ref[pl.ds(c * CHUNK, CHUNK), :] = (
                    g_ref[pl.ds(c * CHUNK, CHUNK), :].astype(f32))
            for _, rdma in inflight:
                rdma.wait_send()
            inflight = nxt

    out_flat = pl.pallas_call(
        body,
        out_shape=jax.ShapeDtypeStruct((ROWS, DM), jnp.float32),
        in_specs=[
            pl.BlockSpec(memory_space=pltpu.VMEM),
            pl.BlockSpec(memory_space=pl.ANY),
            pl.BlockSpec(memory_space=pltpu.VMEM),
            pl.BlockSpec(memory_space=pltpu.VMEM),
            pl.BlockSpec(memory_space=pl.ANY),
        ],
        out_specs=pl.BlockSpec(memory_space=pltpu.VMEM),
        scratch_shapes=[
            pltpu.VMEM((DM, HD), jnp.float32),
            pltpu.VMEM((HD, DM), jnp.float32),
            pltpu.VMEM((ROWS, DM), jnp.bfloat16),
            pltpu.VMEM((ROWS, DM), jnp.bfloat16),
            pltpu.VMEM((N - 1, CHUNK, DM), jnp.bfloat16),
            pltpu.SemaphoreType.DMA((2,)),
            pltpu.SemaphoreType.DMA((N - 1,)),
            pltpu.SemaphoreType.DMA((N - 1,)),
            pltpu.SemaphoreType.DMA((N - 1,)),
            pltpu.SemaphoreType.DMA((N - 1,)),
        ],
        compiler_params=(
            pltpu.CompilerParams(collective_id=0) if DO_COMM
            else pltpu.CompilerParams()),
    )(x, Wq, K_t, V_t, Wo)
    return out_flat.reshape(B, SQ, DM)
